# Initial kernel scaffold; baseline (speedup 1.0000x reference)
import jax
import jax.numpy as jnp
from jax.experimental import pallas as pl


def kernel(r, offsets, idx_ik, idx_jk):
    raise NotImplementedError("write your pallas kernel here")


# speedup vs baseline: 1.9341x; 1.9341x over previous
import functools

import jax
import jax.numpy as jnp
from jax import lax
from jax.experimental import pallas as pl
from jax.experimental.pallas import tpu as pltpu
from jax.experimental.pallas import tpu_sc as plsc

NC = 2
NS = 16
NW = NC * NS
CHUNK = 128


def _newton_sqrt(x):
    xi = lax.bitcast_convert_type(x, jnp.int32)
    yi = jnp.int32(0x5F3759DF) - lax.shift_right_arithmetic(xi, 1)
    y = lax.bitcast_convert_type(yi, jnp.float32)
    half_x = 0.5 * x
    for _ in range(3):
        y = y * (1.5 - half_x * y * y)
    return x * y


def _make_kernel(E, N):
    nchunks = E // CHUNK
    base_trips = nchunks // NW
    extra = nchunks % NW
    mesh = plsc.VectorSubcoreMesh(core_axis_name="c", subcore_axis_name="s")

    @functools.partial(
        pl.kernel,
        out_type=jax.ShapeDtypeStruct((E,), jnp.float32),
        mesh=mesh,
        scratch_types=[
            pltpu.VMEM((CHUNK,), jnp.int32),
            pltpu.VMEM((CHUNK,), jnp.int32),
            pltpu.VMEM((3 * CHUNK,), jnp.float32),
            pltpu.VMEM((CHUNK,), jnp.float32),
            pltpu.VMEM((CHUNK,), jnp.float32),
            pltpu.VMEM((CHUNK,), jnp.float32),
            pltpu.VMEM((CHUNK,), jnp.float32),
            pltpu.VMEM((CHUNK,), jnp.float32),
            pltpu.VMEM((CHUNK,), jnp.float32),
            pltpu.VMEM((CHUNK,), jnp.float32),
            pltpu.SemaphoreType.DMA,
        ],
        compiler_params=pltpu.CompilerParams(needs_layout_passes=False),
    )
    def kern(rx_hbm, ry_hbm, rz_hbm, ii_hbm, ij_hbm, off_hbm, out_hbm,
             ii_v, ij_v, off_v,
             rix_v, riy_v, riz_v, rjx_v, rjy_v, rjz_v, out_v, sem):
        wid = lax.axis_index("s") * NC + lax.axis_index("c")
        ntrips = base_trips + (wid < extra).astype(jnp.int32)

        lanes = lax.iota(jnp.int32, 16)

        def step(t, carry):
            k = wid + NW * t
            base = k * CHUNK
            pltpu.sync_copy(ii_hbm.at[pl.ds(base, CHUNK)], ii_v)
            pltpu.sync_copy(ij_hbm.at[pl.ds(base, CHUNK)], ij_v)
            pltpu.sync_copy(off_hbm.at[pl.ds(base * 3, 3 * CHUNK)], off_v)
            cps = [
                pltpu.async_copy(rx_hbm.at[ii_v], rix_v, sem),
                pltpu.async_copy(ry_hbm.at[ii_v], riy_v, sem),
                pltpu.async_copy(rz_hbm.at[ii_v], riz_v, sem),
                pltpu.async_copy(rx_hbm.at[ij_v], rjx_v, sem),
                pltpu.async_copy(ry_hbm.at[ij_v], rjy_v, sem),
                pltpu.async_copy(rz_hbm.at[ij_v], rjz_v, sem),
            ]
            for cp in cps:
                cp.wait()
            for g in range(CHUNK // 16):
                sl = pl.ds(16 * g, 16)
                e_vec = lanes + (16 * g)
                acc = jnp.zeros((16,), jnp.float32)
                for comp, (iv, jv) in enumerate(
                    ((rix_v, rjx_v), (riy_v, rjy_v), (riz_v, rjz_v))):
                    o = plsc.load_gather(off_v, [e_vec * 3 + comp])
                    d = iv[sl] - jv[sl] - o
                    acc = acc + d * d
                out_v[sl] = _newton_sqrt(acc)
            pltpu.sync_copy(out_v, out_hbm.at[pl.ds(base, CHUNK)])
            return carry

        lax.fori_loop(0, ntrips, step, 0)

    return kern


def kernel(r, offsets, idx_ik, idx_jk):
    B, N, _ = r.shape
    E = idx_ik.shape[1]
    rt = r[0].T
    rx, ry, rz = rt[0], rt[1], rt[2]
    offs = offsets[0].reshape(-1)
    out = _make_kernel(E, N)(rx, ry, rz, idx_ik[0], idx_jk[0], offs)
    return out.reshape(B, E, 1)


# speedup vs baseline: 2.2230x; 1.1494x over previous
import functools

import jax
import jax.numpy as jnp
from jax import lax
from jax.experimental import pallas as pl
from jax.experimental.pallas import tpu as pltpu
from jax.experimental.pallas import tpu_sc as plsc

NC = 2
NS = 16
NW = NC * NS
CHUNK = 512
GB = CHUNK // 128
NEWTON_ITERS = 3


def _newton_sqrt(x):
    xi = lax.bitcast_convert_type(x, jnp.int32)
    yi = jnp.int32(0x5F3759DF) - lax.shift_right_arithmetic(xi, 1)
    y = lax.bitcast_convert_type(yi, jnp.float32)
    half_x = 0.5 * x
    for _ in range(NEWTON_ITERS):
        y = y * (1.5 - half_x * y * y)
    return x * y


def _make_kernel(E, N):
    nchunks = E // CHUNK
    assert nchunks * CHUNK == E
    ntrips_max = -(-nchunks // NW)
    mesh = plsc.VectorSubcoreMesh(core_axis_name="c", subcore_axis_name="s")

    buf = lambda n, dt=jnp.float32: pltpu.VMEM((n,), dt)
    slot_types = [
        buf(CHUNK, jnp.int32),
        buf(CHUNK, jnp.int32),
        buf(3 * CHUNK),
        buf(CHUNK), buf(CHUNK), buf(CHUNK),
        buf(CHUNK), buf(CHUNK), buf(CHUNK),
        buf(CHUNK),
    ]

    @functools.partial(
        pl.kernel,
        out_type=jax.ShapeDtypeStruct((E,), jnp.float32),
        mesh=mesh,
        scratch_types=slot_types + slot_types + [
            pltpu.SemaphoreType.DMA,
            pltpu.SemaphoreType.DMA,
            pltpu.SemaphoreType.DMA,
            pltpu.SemaphoreType.DMA,
            pltpu.SemaphoreType.DMA,
            pltpu.SemaphoreType.DMA,
        ],
        compiler_params=pltpu.CompilerParams(needs_layout_passes=False),
    )
    def kern(rx_hbm, ry_hbm, rz_hbm, ii_hbm, ij_hbm, off_hbm, out_hbm, *rest):
        slots = (rest[0:10], rest[10:20])
        sem_ld = rest[20:22]
        sem_ga = rest[22:24]
        sem_out = rest[24:26]
        wid = lax.axis_index("s") * NC + lax.axis_index("c")
        lanes = lax.iota(jnp.int32, 16)

        def chunk_id(t):
            return wid + NW * t

        def issue_loads(k, p):
            ii_v, ij_v, off_v = slots[p][0], slots[p][1], slots[p][2]
            base = k * CHUNK
            pltpu.async_copy(ii_hbm.at[pl.ds(base, CHUNK)], ii_v, sem_ld[p])
            pltpu.async_copy(ij_hbm.at[pl.ds(base, CHUNK)], ij_v, sem_ld[p])
            pltpu.async_copy(off_hbm.at[pl.ds(base * 3, 3 * CHUNK)], off_v,
                             sem_ld[p])

        def wait_loads(p):
            ii_v, ij_v, off_v = slots[p][0], slots[p][1], slots[p][2]
            pltpu.make_async_copy(ii_hbm.at[pl.ds(0, CHUNK)], ii_v,
                                  sem_ld[p]).wait()
            pltpu.make_async_copy(ij_hbm.at[pl.ds(0, CHUNK)], ij_v,
                                  sem_ld[p]).wait()
            pltpu.make_async_copy(off_hbm.at[pl.ds(0, 3 * CHUNK)], off_v,
                                  sem_ld[p]).wait()

        def issue_gathers(p):
            (ii_v, ij_v, _off, rix, riy, riz, rjx, rjy, rjz, _o) = slots[p]
            for b in range(GB):
                s = pl.ds(128 * b, 128)
                for tab, idx_v, dst in ((rx_hbm, ii_v, rix),
                                        (ry_hbm, ii_v, riy),
                                        (rz_hbm, ii_v, riz),
                                        (rx_hbm, ij_v, rjx),
                                        (ry_hbm, ij_v, rjy),
                                        (rz_hbm, ij_v, rjz)):
                    pltpu.async_copy(tab.at[idx_v.at[s]], dst.at[s], sem_ga[p])

        def wait_gathers(p):
            (_, _, _off, rix, riy, riz, rjx, rjy, rjz, _o) = slots[p]
            for b in range(GB):
                s = pl.ds(128 * b, 128)
                for tab, dst in ((rx_hbm, rix), (ry_hbm, riy), (rz_hbm, riz),
                                 (rx_hbm, rjx), (ry_hbm, rjy), (rz_hbm, rjz)):
                    pltpu.make_async_copy(tab.at[pl.ds(0, 128)], dst.at[s],
                                          sem_ga[p]).wait()

        def compute(k, p):
            (_, _, off_v, rix, riy, riz, rjx, rjy, rjz, out_v) = slots[p]
            for g in range(CHUNK // 16):
                sl = pl.ds(16 * g, 16)
                e3 = (lanes + 16 * g) * 3
                acc = jnp.zeros((16,), jnp.float32)
                for comp, (iv, jv) in enumerate(
                        ((rix, rjx), (riy, rjy), (riz, rjz))):
                    o = plsc.load_gather(off_v, [e3 + comp])
                    d = iv[sl] - jv[sl] - o
                    acc = acc + d * d
                out_v[sl] = _newton_sqrt(acc)
            pltpu.async_copy(out_v, out_hbm.at[pl.ds(k * CHUNK, CHUNK)],
                             sem_out[p])

        def wait_out(p):
            out_v = slots[p][9]
            pltpu.make_async_copy(out_hbm.at[pl.ds(0, CHUNK)], out_v,
                                  sem_out[p]).wait()

        issue_loads(chunk_id(0), 0)
        wait_loads(0)
        issue_gathers(0)

        def do_trip(t, p):
            k = chunk_id(t)
            knext = chunk_id(t + 1)
            nvalid = knext < nchunks

            @pl.when(nvalid)
            def _():
                issue_loads(knext, 1 - p)

            wait_gathers(p)

            @pl.when(nvalid)
            def _():
                wait_loads(1 - p)
                issue_gathers(1 - p)

            @pl.when(t >= 2)
            def _():
                wait_out(p)
            compute(k, p)

        def body(u, carry):
            t0 = u * 2

            @pl.when(chunk_id(t0) < nchunks)
            def _():
                do_trip(t0, 0)

            @pl.when(chunk_id(t0 + 1) < nchunks)
            def _():
                do_trip(t0 + 1, 1)
            return carry

        lax.fori_loop(0, (ntrips_max + 1) // 2, body, 0)
        pltpu.make_async_copy(out_hbm.at[pl.ds(0, CHUNK)], slots[0][9],
                              sem_out[0]).wait()
        pltpu.make_async_copy(out_hbm.at[pl.ds(0, CHUNK)], slots[1][9],
                              sem_out[1]).wait()

    return kern


def kernel(r, offsets, idx_ik, idx_jk):
    B, N, _ = r.shape
    E = idx_ik.shape[1]
    rt = r[0].T
    rx, ry, rz = rt[0], rt[1], rt[2]
    offs = offsets[0].reshape(-1)
    out = _make_kernel(E, N)(rx, ry, rz, idx_ik[0], idx_jk[0], offs)
    return out.reshape(B, E, 1)


# speedup vs baseline: 27.9105x; 12.5551x over previous
import functools

import jax
import jax.numpy as jnp
from jax import lax
from jax.experimental import pallas as pl
from jax.experimental.pallas import tpu as pltpu
from jax.experimental.pallas import tpu_sc as plsc

NC = 2
NS = 16
NW = NC * NS
CHUNK = 512
GB = CHUNK // 128
NEWTON_ITERS = 3


def _newton_sqrt(x):
    xi = lax.bitcast_convert_type(x, jnp.int32)
    yi = jnp.int32(0x5F3759DF) - lax.shift_right_arithmetic(xi, 1)
    y = lax.bitcast_convert_type(yi, jnp.float32)
    half_x = 0.5 * x
    for _ in range(NEWTON_ITERS):
        y = y * (1.5 - half_x * y * y)
    return x * y


def _make_kernel(E, N):
    nchunks = E // CHUNK
    assert nchunks * CHUNK == E
    ntrips_max = -(-nchunks // NW)
    mesh = plsc.VectorSubcoreMesh(core_axis_name="c", subcore_axis_name="s")

    buf = lambda n, dt=jnp.float32: pltpu.VMEM((n,), dt)
    slot_types = [
        buf(CHUNK, jnp.int32),
        buf(CHUNK, jnp.int32),
        buf(CHUNK), buf(CHUNK), buf(CHUNK),
        buf(CHUNK), buf(CHUNK), buf(CHUNK),
        buf(CHUNK), buf(CHUNK), buf(CHUNK),
        buf(CHUNK),
    ]

    @functools.partial(
        pl.kernel,
        out_type=jax.ShapeDtypeStruct((E,), jnp.float32),
        mesh=mesh,
        scratch_types=slot_types + slot_types + [
            pltpu.SemaphoreType.DMA,
            pltpu.SemaphoreType.DMA,
            pltpu.SemaphoreType.DMA,
            pltpu.SemaphoreType.DMA,
            pltpu.SemaphoreType.DMA,
            pltpu.SemaphoreType.DMA,
        ],
        compiler_params=pltpu.CompilerParams(needs_layout_passes=False),
    )
    def kern(rx_hbm, ry_hbm, rz_hbm, ii_hbm, ij_hbm,
             ox_hbm, oy_hbm, oz_hbm, out_hbm, *rest):
        slots = (rest[0:12], rest[12:24])
        sem_ld = rest[24:26]
        sem_ga = rest[26:28]
        sem_out = rest[28:30]
        wid = lax.axis_index("s") * NC + lax.axis_index("c")

        def chunk_id(t):
            return wid + NW * t

        def issue_loads(k, p):
            ii_v, ij_v, ox_v, oy_v, oz_v = slots[p][0:5]
            base = k * CHUNK
            sl = pl.ds(base, CHUNK)
            pltpu.async_copy(ii_hbm.at[sl], ii_v, sem_ld[p])
            pltpu.async_copy(ij_hbm.at[sl], ij_v, sem_ld[p])
            pltpu.async_copy(ox_hbm.at[sl], ox_v, sem_ld[p])
            pltpu.async_copy(oy_hbm.at[sl], oy_v, sem_ld[p])
            pltpu.async_copy(oz_hbm.at[sl], oz_v, sem_ld[p])

        def wait_loads(p):
            for dst in slots[p][0:5]:
                pltpu.make_async_copy(out_hbm.at[pl.ds(0, CHUNK)], dst,
                                      sem_ld[p]).wait()

        def issue_gathers(p):
            (ii_v, ij_v, _ox, _oy, _oz,
             rix, riy, riz, rjx, rjy, rjz, _o) = slots[p]
            for b in range(GB):
                s = pl.ds(128 * b, 128)
                for tab, idx_v, dst in ((rx_hbm, ii_v, rix),
                                        (ry_hbm, ii_v, riy),
                                        (rz_hbm, ii_v, riz),
                                        (rx_hbm, ij_v, rjx),
                                        (ry_hbm, ij_v, rjy),
                                        (rz_hbm, ij_v, rjz)):
                    pltpu.async_copy(tab.at[idx_v.at[s]], dst.at[s], sem_ga[p])

        def wait_gathers(p):
            dsts = slots[p][5:11]
            for b in range(GB):
                s = pl.ds(128 * b, 128)
                for dst in dsts:
                    pltpu.make_async_copy(out_hbm.at[pl.ds(0, 128)],
                                          dst.at[s], sem_ga[p]).wait()

        def compute(k, p):
            (_, _, ox_v, oy_v, oz_v,
             rix, riy, riz, rjx, rjy, rjz, out_v) = slots[p]
            for g in range(CHUNK // 16):
                sl = pl.ds(16 * g, 16)
                acc = jnp.zeros((16,), jnp.float32)
                for iv, jv, ov in ((rix, rjx, ox_v),
                                   (riy, rjy, oy_v),
                                   (riz, rjz, oz_v)):
                    d = iv[sl] - jv[sl] - ov[sl]
                    acc = acc + d * d
                out_v[sl] = _newton_sqrt(acc)
            pltpu.async_copy(out_v, out_hbm.at[pl.ds(k * CHUNK, CHUNK)],
                             sem_out[p])

        def wait_out(p):
            out_v = slots[p][11]
            pltpu.make_async_copy(out_hbm.at[pl.ds(0, CHUNK)], out_v,
                                  sem_out[p]).wait()

        issue_loads(chunk_id(0), 0)
        wait_loads(0)
        issue_gathers(0)

        def do_trip(t, p):
            k = chunk_id(t)
            knext = chunk_id(t + 1)
            nvalid = knext < nchunks

            @pl.when(nvalid)
            def _():
                issue_loads(knext, 1 - p)

            wait_gathers(p)

            @pl.when(nvalid)
            def _():
                wait_loads(1 - p)
                issue_gathers(1 - p)

            @pl.when(t >= 2)
            def _():
                wait_out(p)
            compute(k, p)

        def body(u, carry):
            t0 = u * 2

            @pl.when(chunk_id(t0) < nchunks)
            def _():
                do_trip(t0, 0)

            @pl.when(chunk_id(t0 + 1) < nchunks)
            def _():
                do_trip(t0 + 1, 1)
            return carry

        lax.fori_loop(0, (ntrips_max + 1) // 2, body, 0)
        pltpu.make_async_copy(out_hbm.at[pl.ds(0, CHUNK)], slots[0][11],
                              sem_out[0]).wait()
        pltpu.make_async_copy(out_hbm.at[pl.ds(0, CHUNK)], slots[1][11],
                              sem_out[1]).wait()

    return kern


def kernel(r, offsets, idx_ik, idx_jk):
    B, N, _ = r.shape
    E = idx_ik.shape[1]
    out = _make_kernel(E, N)(r[0, :, 0], r[0, :, 1], r[0, :, 2],
                             idx_ik[0], idx_jk[0],
                             offsets[0, :, 0], offsets[0, :, 1],
                             offsets[0, :, 2])
    return out.reshape(B, E, 1)


# speedup vs baseline: 28.0745x; 1.0059x over previous
import functools

import jax
import jax.numpy as jnp
from jax import lax
from jax.experimental import pallas as pl
from jax.experimental.pallas import tpu as pltpu
from jax.experimental.pallas import tpu_sc as plsc

NC = 2
NS = 16
NW = NC * NS
CHUNK = 512
GB = CHUNK // 128
NEWTON_ITERS = 3


def _newton_sqrt(x):
    xi = lax.bitcast_convert_type(x, jnp.int32)
    yi = jnp.int32(0x5F3759DF) - lax.shift_right_arithmetic(xi, 1)
    y = lax.bitcast_convert_type(yi, jnp.float32)
    half_x = 0.5 * x
    for _ in range(NEWTON_ITERS):
        y = y * (1.5 - half_x * y * y)
    return x * y


def _make_kernel(E, N):
    nchunks = E // CHUNK
    assert nchunks * CHUNK == E
    ntrips_max = -(-nchunks // NW)
    mesh = plsc.VectorSubcoreMesh(core_axis_name="c", subcore_axis_name="s")

    buf = lambda n, dt=jnp.float32: pltpu.VMEM((n,), dt)
    slot_types = [
        buf(CHUNK, jnp.int32),
        buf(CHUNK, jnp.int32),
        buf(CHUNK), buf(CHUNK), buf(CHUNK),
        buf(CHUNK), buf(CHUNK), buf(CHUNK),
        buf(CHUNK), buf(CHUNK), buf(CHUNK),
        buf(CHUNK),
    ]

    @functools.partial(
        pl.kernel,
        out_type=jax.ShapeDtypeStruct((E,), jnp.float32),
        mesh=mesh,
        scratch_types=slot_types + slot_types + [
            pltpu.SemaphoreType.DMA,
            pltpu.SemaphoreType.DMA,
            pltpu.SemaphoreType.DMA,
            pltpu.SemaphoreType.DMA,
            pltpu.SemaphoreType.DMA,
            pltpu.SemaphoreType.DMA,
        ],
        compiler_params=pltpu.CompilerParams(needs_layout_passes=False),
    )
    def kern(rx_hbm, ry_hbm, rz_hbm, ii_hbm, ij_hbm,
             ox_hbm, oy_hbm, oz_hbm, out_hbm, *rest):
        slots = (rest[0:12], rest[12:24])
        sem_ld = rest[24:26]
        sem_ga = rest[26:28]
        sem_out = rest[28:30]
        wid = lax.axis_index("s") * NC + lax.axis_index("c")

        def chunk_id(t):
            return wid + NW * t

        def issue_loads(k, p):
            ii_v, ij_v, ox_v, oy_v, oz_v = slots[p][0:5]
            base = k * CHUNK
            sl = pl.ds(base, CHUNK)
            pltpu.async_copy(ii_hbm.at[sl], ii_v, sem_ld[p])
            pltpu.async_copy(ij_hbm.at[sl], ij_v, sem_ld[p])
            pltpu.async_copy(ox_hbm.at[sl], ox_v, sem_ld[p])
            pltpu.async_copy(oy_hbm.at[sl], oy_v, sem_ld[p])
            pltpu.async_copy(oz_hbm.at[sl], oz_v, sem_ld[p])

        def wait_loads(p):
            for dst in slots[p][0:5]:
                pltpu.make_async_copy(out_hbm.at[pl.ds(0, CHUNK)], dst,
                                      sem_ld[p]).wait()

        def issue_gathers(p):
            (ii_v, ij_v, _ox, _oy, _oz,
             rix, riy, riz, rjx, rjy, rjz, _o) = slots[p]
            for tab, idx_v, dst in ((rx_hbm, ii_v, rix),
                                    (ry_hbm, ii_v, riy),
                                    (rz_hbm, ii_v, riz),
                                    (rx_hbm, ij_v, rjx),
                                    (ry_hbm, ij_v, rjy),
                                    (rz_hbm, ij_v, rjz)):
                pltpu.async_copy(tab.at[idx_v], dst, sem_ga[p])

        def wait_gathers(p):
            for dst in slots[p][5:11]:
                pltpu.make_async_copy(out_hbm.at[pl.ds(0, CHUNK)],
                                      dst, sem_ga[p]).wait()

        def compute(k, p):
            (_, _, ox_v, oy_v, oz_v,
             rix, riy, riz, rjx, rjy, rjz, out_v) = slots[p]
            for g in range(CHUNK // 16):
                sl = pl.ds(16 * g, 16)
                acc = jnp.zeros((16,), jnp.float32)
                for iv, jv, ov in ((rix, rjx, ox_v),
                                   (riy, rjy, oy_v),
                                   (riz, rjz, oz_v)):
                    d = iv[sl] - jv[sl] - ov[sl]
                    acc = acc + d * d
                out_v[sl] = _newton_sqrt(acc)
            pltpu.async_copy(out_v, out_hbm.at[pl.ds(k * CHUNK, CHUNK)],
                             sem_out[p])

        def wait_out(p):
            out_v = slots[p][11]
            pltpu.make_async_copy(out_hbm.at[pl.ds(0, CHUNK)], out_v,
                                  sem_out[p]).wait()

        issue_loads(chunk_id(0), 0)
        wait_loads(0)
        issue_gathers(0)

        def do_trip(t, p):
            k = chunk_id(t)
            knext = chunk_id(t + 1)
            nvalid = knext < nchunks

            @pl.when(nvalid)
            def _():
                issue_loads(knext, 1 - p)

            wait_gathers(p)

            @pl.when(nvalid)
            def _():
                wait_loads(1 - p)
                issue_gathers(1 - p)

            @pl.when(t >= 2)
            def _():
                wait_out(p)
            compute(k, p)

        def body(u, carry):
            t0 = u * 2

            @pl.when(chunk_id(t0) < nchunks)
            def _():
                do_trip(t0, 0)

            @pl.when(chunk_id(t0 + 1) < nchunks)
            def _():
                do_trip(t0 + 1, 1)
            return carry

        lax.fori_loop(0, (ntrips_max + 1) // 2, body, 0)
        pltpu.make_async_copy(out_hbm.at[pl.ds(0, CHUNK)], slots[0][11],
                              sem_out[0]).wait()
        pltpu.make_async_copy(out_hbm.at[pl.ds(0, CHUNK)], slots[1][11],
                              sem_out[1]).wait()

    return kern


def kernel(r, offsets, idx_ik, idx_jk):
    B, N, _ = r.shape
    E = idx_ik.shape[1]
    out = _make_kernel(E, N)(r[0, :, 0], r[0, :, 1], r[0, :, 2],
                             idx_ik[0], idx_jk[0],
                             offsets[0, :, 0], offsets[0, :, 1],
                             offsets[0, :, 2])
    return out.reshape(B, E, 1)


# speedup vs baseline: 64.1367x; 2.2845x over previous
import functools

import jax
import jax.numpy as jnp
from jax import lax
from jax.experimental import pallas as pl
from jax.experimental.pallas import tpu as pltpu
from jax.experimental.pallas import tpu_sc as plsc

NC = 2
NS = 16
NW = NC * NS
CHUNK = 512
NEWTON_ITERS = 3
STAGE_BLK = 10000


def _newton_sqrt(x):
    xi = lax.bitcast_convert_type(x, jnp.int32)
    yi = jnp.int32(0x5F3759DF) - lax.shift_right_arithmetic(xi, 1)
    y = lax.bitcast_convert_type(yi, jnp.float32)
    half_x = 0.5 * x
    for _ in range(NEWTON_ITERS):
        y = y * (1.5 - half_x * y * y)
    return x * y


def _make_kernel(E, N):
    nchunks = E // CHUNK
    assert nchunks * CHUNK == E
    ntrips_max = -(-nchunks // NW)
    nblk = -(-N // STAGE_BLK)
    mesh = plsc.VectorSubcoreMesh(core_axis_name="c", subcore_axis_name="s")

    buf = lambda n, dt=jnp.float32: pltpu.VMEM((n,), dt)
    slot_types = [
        buf(CHUNK, jnp.int32),
        buf(CHUNK, jnp.int32),
        buf(CHUNK), buf(CHUNK), buf(CHUNK),
        buf(CHUNK), buf(CHUNK), buf(CHUNK),
        buf(CHUNK), buf(CHUNK), buf(CHUNK),
        buf(CHUNK),
        buf(CHUNK, jnp.int32), buf(CHUNK, jnp.int32), buf(CHUNK, jnp.int32),
        buf(CHUNK, jnp.int32), buf(CHUNK, jnp.int32), buf(CHUNK, jnp.int32),
    ]

    @functools.partial(
        pl.kernel,
        out_type=jax.ShapeDtypeStruct((E,), jnp.float32),
        mesh=mesh,
        scratch_types=slot_types + slot_types + [
            pltpu.SemaphoreType.DMA,
            pltpu.SemaphoreType.DMA,
            pltpu.SemaphoreType.DMA,
            pltpu.SemaphoreType.DMA,
            pltpu.SemaphoreType.DMA,
            pltpu.SemaphoreType.DMA,
            pltpu.VMEM_SHARED((4 * N,), jnp.float32),
            buf(STAGE_BLK), buf(STAGE_BLK), buf(STAGE_BLK),
            buf(4 * STAGE_BLK),
        ],
        compiler_params=pltpu.CompilerParams(needs_layout_passes=False),
    )
    def kern(rx_hbm, ry_hbm, rz_hbm, ii_hbm, ij_hbm,
             ox_hbm, oy_hbm, oz_hbm, out_hbm, *rest):
        slots = (rest[0:18], rest[18:36])
        sem_ld = rest[36:38]
        sem_ga = rest[38:40]
        sem_out = rest[40:42]
        r4_sh = rest[42]
        sx_v, sy_v, sz_v, st_v = rest[43:47]
        sid = lax.axis_index("s")
        wid = sid * NC + lax.axis_index("c")
        lanes = lax.iota(jnp.int32, 16)

        @pl.when(sid == 0)
        def _():
            def stage_blk(b, carry):
                start = b * STAGE_BLK
                sl_src = pl.ds(start, STAGE_BLK)
                pltpu.sync_copy(rx_hbm.at[sl_src], sx_v)
                pltpu.sync_copy(ry_hbm.at[sl_src], sy_v)
                pltpu.sync_copy(rz_hbm.at[sl_src], sz_v)
                for g in range(STAGE_BLK // 16):
                    sl = pl.ds(16 * g, 16)
                    tgt = lanes * 4 + (64 * g)
                    plsc.store_scatter(st_v, [tgt], sx_v[sl])
                    plsc.store_scatter(st_v, [tgt + 1], sy_v[sl])
                    plsc.store_scatter(st_v, [tgt + 2], sz_v[sl])
                pltpu.sync_copy(st_v, r4_sh.at[pl.ds(4 * start, 4 * STAGE_BLK)])
                return carry

            lax.fori_loop(0, nblk, stage_blk, 0)

        plsc.subcore_barrier()

        def chunk_id(t):
            return wid + NW * t

        def issue_loads(k, p):
            ii_v, ij_v, ox_v, oy_v, oz_v = slots[p][0:5]
            base = k * CHUNK
            sl = pl.ds(base, CHUNK)
            pltpu.async_copy(ii_hbm.at[sl], ii_v, sem_ld[p])
            pltpu.async_copy(ij_hbm.at[sl], ij_v, sem_ld[p])
            pltpu.async_copy(ox_hbm.at[sl], ox_v, sem_ld[p])
            pltpu.async_copy(oy_hbm.at[sl], oy_v, sem_ld[p])
            pltpu.async_copy(oz_hbm.at[sl], oz_v, sem_ld[p])

        def wait_loads(p):
            for dst in slots[p][0:5]:
                pltpu.make_async_copy(out_hbm.at[pl.ds(0, CHUNK)], dst,
                                      sem_ld[p]).wait()

        def build_indices(p):
            ii_v, ij_v = slots[p][0:2]
            (wxi, wyi, wzi, wxj, wyj, wzj) = slots[p][12:18]
            for g in range(CHUNK // 16):
                sl = pl.ds(16 * g, 16)
                bi = lax.shift_left(ii_v[sl], 2)
                wxi[sl] = bi
                wyi[sl] = bi + 1
                wzi[sl] = bi + 2
                bj = lax.shift_left(ij_v[sl], 2)
                wxj[sl] = bj
                wyj[sl] = bj + 1
                wzj[sl] = bj + 2

        def issue_gathers(p):
            (rix, riy, riz, rjx, rjy, rjz) = slots[p][5:11]
            (wxi, wyi, wzi, wxj, wyj, wzj) = slots[p][12:18]
            for idx_v, dst in ((wxi, rix), (wyi, riy), (wzi, riz),
                               (wxj, rjx), (wyj, rjy), (wzj, rjz)):
                pltpu.async_copy(r4_sh.at[idx_v], dst, sem_ga[p])

        def wait_gathers(p):
            for dst in slots[p][5:11]:
                pltpu.make_async_copy(out_hbm.at[pl.ds(0, CHUNK)],
                                      dst, sem_ga[p]).wait()

        def compute(k, p):
            (_, _, ox_v, oy_v, oz_v,
             rix, riy, riz, rjx, rjy, rjz, out_v) = slots[p][0:12]
            for g in range(CHUNK // 16):
                sl = pl.ds(16 * g, 16)
                acc = jnp.zeros((16,), jnp.float32)
                for iv, jv, ov in ((rix, rjx, ox_v),
                                   (riy, rjy, oy_v),
                                   (riz, rjz, oz_v)):
                    d = iv[sl] - jv[sl] - ov[sl]
                    acc = acc + d * d
                out_v[sl] = _newton_sqrt(acc)
            pltpu.async_copy(out_v, out_hbm.at[pl.ds(k * CHUNK, CHUNK)],
                             sem_out[p])

        def wait_out(p):
            out_v = slots[p][11]
            pltpu.make_async_copy(out_hbm.at[pl.ds(0, CHUNK)], out_v,
                                  sem_out[p]).wait()

        issue_loads(chunk_id(0), 0)
        wait_loads(0)
        build_indices(0)
        issue_gathers(0)

        def do_trip(t, p):
            k = chunk_id(t)
            knext = chunk_id(t + 1)
            nvalid = knext < nchunks

            @pl.when(nvalid)
            def _():
                issue_loads(knext, 1 - p)

            wait_gathers(p)

            @pl.when(nvalid)
            def _():
                wait_loads(1 - p)
                build_indices(1 - p)
                issue_gathers(1 - p)

            @pl.when(t >= 2)
            def _():
                wait_out(p)
            compute(k, p)

        def body(u, carry):
            t0 = u * 2

            @pl.when(chunk_id(t0) < nchunks)
            def _():
                do_trip(t0, 0)

            @pl.when(chunk_id(t0 + 1) < nchunks)
            def _():
                do_trip(t0 + 1, 1)
            return carry

        lax.fori_loop(0, (ntrips_max + 1) // 2, body, 0)
        pltpu.make_async_copy(out_hbm.at[pl.ds(0, CHUNK)], slots[0][11],
                              sem_out[0]).wait()
        pltpu.make_async_copy(out_hbm.at[pl.ds(0, CHUNK)], slots[1][11],
                              sem_out[1]).wait()

    return kern


def kernel(r, offsets, idx_ik, idx_jk):
    B, N, _ = r.shape
    E = idx_ik.shape[1]
    out = _make_kernel(E, N)(r[0, :, 0], r[0, :, 1], r[0, :, 2],
                             idx_ik[0], idx_jk[0],
                             offsets[0, :, 0], offsets[0, :, 1],
                             offsets[0, :, 2])
    return out.reshape(B, E, 1)


# speedup vs baseline: 87.6572x; 1.3667x over previous
import functools

import jax
import jax.numpy as jnp
from jax import lax
from jax.experimental import pallas as pl
from jax.experimental.pallas import tpu as pltpu
from jax.experimental.pallas import tpu_sc as plsc

NC = 2
NS = 16
NW = NC * NS
CHUNK = 512
NEWTON_ITERS = 3
STAGE_PTS = 6256


def _newton_sqrt(x):
    xi = lax.bitcast_convert_type(x, jnp.int32)
    yi = jnp.int32(0x5F3759DF) - lax.shift_right_arithmetic(xi, 1)
    y = lax.bitcast_convert_type(yi, jnp.float32)
    half_x = 0.5 * x
    for _ in range(NEWTON_ITERS):
        y = y * (1.5 - half_x * y * y)
    return x * y


def _make_kernel(E, N):
    nchunks = E // CHUNK
    assert nchunks * CHUNK == E
    ntrips_max = -(-nchunks // NW)
    stage_tail = N - (NS - 1) * STAGE_PTS
    assert 0 < stage_tail <= STAGE_PTS and stage_tail % 16 == 0
    mesh = plsc.VectorSubcoreMesh(core_axis_name="c", subcore_axis_name="s")

    buf = lambda n, dt=jnp.float32: pltpu.VMEM((n,), dt)
    slot_types = [
        buf(CHUNK, jnp.int32),
        buf(CHUNK, jnp.int32),
        buf(CHUNK), buf(CHUNK), buf(CHUNK),
        buf(CHUNK), buf(CHUNK), buf(CHUNK),
        buf(CHUNK), buf(CHUNK), buf(CHUNK),
        buf(CHUNK),
        buf(CHUNK, jnp.int32), buf(CHUNK, jnp.int32), buf(CHUNK, jnp.int32),
        buf(CHUNK, jnp.int32), buf(CHUNK, jnp.int32), buf(CHUNK, jnp.int32),
    ]

    @functools.partial(
        pl.kernel,
        out_type=jax.ShapeDtypeStruct((E,), jnp.float32),
        mesh=mesh,
        scratch_types=slot_types + slot_types + [
            pltpu.SemaphoreType.DMA,
            pltpu.SemaphoreType.DMA,
            pltpu.SemaphoreType.DMA,
            pltpu.SemaphoreType.DMA,
            pltpu.SemaphoreType.DMA,
            pltpu.SemaphoreType.DMA,
            pltpu.VMEM_SHARED((4 * N,), jnp.float32),
            buf(STAGE_PTS), buf(STAGE_PTS), buf(STAGE_PTS),
            buf(4 * STAGE_PTS),
        ],
        compiler_params=pltpu.CompilerParams(needs_layout_passes=False),
    )
    def kern(rx_hbm, ry_hbm, rz_hbm, ii_hbm, ij_hbm,
             ox_hbm, oy_hbm, oz_hbm, out_hbm, *rest):
        slots = (rest[0:18], rest[18:36])
        sem_ld = rest[36:38]
        sem_ga = rest[38:40]
        sem_out = rest[40:42]
        r4_sh = rest[42]
        sx_v, sy_v, sz_v, st_v = rest[43:47]
        sid = lax.axis_index("s")
        wid = sid * NC + lax.axis_index("c")
        lanes = lax.iota(jnp.int32, 16)

        def stage(npts):
            start = sid * STAGE_PTS
            sl_src = pl.ds(start, npts)
            pltpu.sync_copy(rx_hbm.at[sl_src], sx_v.at[pl.ds(0, npts)])
            pltpu.sync_copy(ry_hbm.at[sl_src], sy_v.at[pl.ds(0, npts)])
            pltpu.sync_copy(rz_hbm.at[sl_src], sz_v.at[pl.ds(0, npts)])
            for g in range(npts // 16):
                sl = pl.ds(16 * g, 16)
                tgt = lanes * 4 + (64 * g)
                plsc.store_scatter(st_v, [tgt], sx_v[sl])
                plsc.store_scatter(st_v, [tgt + 1], sy_v[sl])
                plsc.store_scatter(st_v, [tgt + 2], sz_v[sl])
            pltpu.sync_copy(st_v.at[pl.ds(0, 4 * npts)],
                            r4_sh.at[pl.ds(4 * start, 4 * npts)])

        @pl.when(sid < NS - 1)
        def _():
            stage(STAGE_PTS)

        @pl.when(sid == NS - 1)
        def _():
            stage(stage_tail)

        plsc.subcore_barrier()

        def chunk_id(t):
            return wid + NW * t

        def issue_loads(k, p):
            ii_v, ij_v, ox_v, oy_v, oz_v = slots[p][0:5]
            base = k * CHUNK
            sl = pl.ds(base, CHUNK)
            pltpu.async_copy(ii_hbm.at[sl], ii_v, sem_ld[p])
            pltpu.async_copy(ij_hbm.at[sl], ij_v, sem_ld[p])
            pltpu.async_copy(ox_hbm.at[sl], ox_v, sem_ld[p])
            pltpu.async_copy(oy_hbm.at[sl], oy_v, sem_ld[p])
            pltpu.async_copy(oz_hbm.at[sl], oz_v, sem_ld[p])

        def wait_loads(p):
            for dst in slots[p][0:5]:
                pltpu.make_async_copy(out_hbm.at[pl.ds(0, CHUNK)], dst,
                                      sem_ld[p]).wait()

        def build_indices(p):
            ii_v, ij_v = slots[p][0:2]
            (wxi, wyi, wzi, wxj, wyj, wzj) = slots[p][12:18]
            for g in range(CHUNK // 16):
                sl = pl.ds(16 * g, 16)
                bi = lax.shift_left(ii_v[sl], 2)
                wxi[sl] = bi
                wyi[sl] = bi + 1
                wzi[sl] = bi + 2
                bj = lax.shift_left(ij_v[sl], 2)
                wxj[sl] = bj
                wyj[sl] = bj + 1
                wzj[sl] = bj + 2

        def issue_gathers(p):
            (rix, riy, riz, rjx, rjy, rjz) = slots[p][5:11]
            (wxi, wyi, wzi, wxj, wyj, wzj) = slots[p][12:18]
            for idx_v, dst in ((wxi, rix), (wyi, riy), (wzi, riz),
                               (wxj, rjx), (wyj, rjy), (wzj, rjz)):
                pltpu.async_copy(r4_sh.at[idx_v], dst, sem_ga[p])

        def wait_gathers(p):
            for dst in slots[p][5:11]:
                pltpu.make_async_copy(out_hbm.at[pl.ds(0, CHUNK)],
                                      dst, sem_ga[p]).wait()

        def compute(k, p):
            (_, _, ox_v, oy_v, oz_v,
             rix, riy, riz, rjx, rjy, rjz, out_v) = slots[p][0:12]
            for g in range(CHUNK // 16):
                sl = pl.ds(16 * g, 16)
                acc = jnp.zeros((16,), jnp.float32)
                for iv, jv, ov in ((rix, rjx, ox_v),
                                   (riy, rjy, oy_v),
                                   (riz, rjz, oz_v)):
                    d = iv[sl] - jv[sl] - ov[sl]
                    acc = acc + d * d
                out_v[sl] = _newton_sqrt(acc)
            pltpu.async_copy(out_v, out_hbm.at[pl.ds(k * CHUNK, CHUNK)],
                             sem_out[p])

        def wait_out(p):
            out_v = slots[p][11]
            pltpu.make_async_copy(out_hbm.at[pl.ds(0, CHUNK)], out_v,
                                  sem_out[p]).wait()

        issue_loads(chunk_id(0), 0)
        wait_loads(0)
        build_indices(0)
        issue_gathers(0)

        def do_trip(t, p):
            k = chunk_id(t)
            knext = chunk_id(t + 1)
            nvalid = knext < nchunks

            @pl.when(nvalid)
            def _():
                issue_loads(knext, 1 - p)

            wait_gathers(p)

            @pl.when(nvalid)
            def _():
                wait_loads(1 - p)
                build_indices(1 - p)
                issue_gathers(1 - p)

            @pl.when(t >= 2)
            def _():
                wait_out(p)
            compute(k, p)

        def body(u, carry):
            t0 = u * 2

            @pl.when(chunk_id(t0) < nchunks)
            def _():
                do_trip(t0, 0)

            @pl.when(chunk_id(t0 + 1) < nchunks)
            def _():
                do_trip(t0 + 1, 1)
            return carry

        lax.fori_loop(0, (ntrips_max + 1) // 2, body, 0)
        pltpu.make_async_copy(out_hbm.at[pl.ds(0, CHUNK)], slots[0][11],
                              sem_out[0]).wait()
        pltpu.make_async_copy(out_hbm.at[pl.ds(0, CHUNK)], slots[1][11],
                              sem_out[1]).wait()

    return kern


def kernel(r, offsets, idx_ik, idx_jk):
    B, N, _ = r.shape
    E = idx_ik.shape[1]
    out = _make_kernel(E, N)(r[0, :, 0], r[0, :, 1], r[0, :, 2],
                             idx_ik[0], idx_jk[0],
                             offsets[0, :, 0], offsets[0, :, 1],
                             offsets[0, :, 2])
    return out.reshape(B, E, 1)


# speedup vs baseline: 98.5499x; 1.1243x over previous
import functools

import jax
import jax.numpy as jnp
from jax import lax
from jax.experimental import pallas as pl
from jax.experimental.pallas import tpu as pltpu
from jax.experimental.pallas import tpu_sc as plsc

NC = 2
NS = 16
NW = NC * NS
CHUNK = 512
NEWTON_ITERS = 2
STAGE_PTS = 6256


def _newton_sqrt(x):
    xi = lax.bitcast_convert_type(x, jnp.int32)
    yi = jnp.int32(0x5F3759DF) - lax.shift_right_arithmetic(xi, 1)
    y = lax.bitcast_convert_type(yi, jnp.float32)
    half_x = 0.5 * x
    for _ in range(NEWTON_ITERS):
        y = y * (1.5 - half_x * y * y)
    return x * y


def _make_kernel(E, N):
    nchunks = E // CHUNK
    assert nchunks * CHUNK == E
    ntrips_max = -(-nchunks // NW)
    stage_tail = N - (NS - 1) * STAGE_PTS
    assert 0 < stage_tail <= STAGE_PTS
    mesh = plsc.VectorSubcoreMesh(core_axis_name="c", subcore_axis_name="s")

    buf = lambda n, dt=jnp.float32: pltpu.VMEM((n,), dt)
    slot_types = [
        buf(CHUNK, jnp.int32),
        buf(CHUNK, jnp.int32),
        buf(CHUNK), buf(CHUNK), buf(CHUNK),
        buf(CHUNK), buf(CHUNK), buf(CHUNK),
        buf(CHUNK), buf(CHUNK), buf(CHUNK),
        buf(CHUNK),
    ]

    @functools.partial(
        pl.kernel,
        out_type=jax.ShapeDtypeStruct((E,), jnp.float32),
        mesh=mesh,
        scratch_types=slot_types + slot_types + [
            pltpu.SemaphoreType.DMA,
            pltpu.SemaphoreType.DMA,
            pltpu.SemaphoreType.DMA,
            pltpu.SemaphoreType.DMA,
            pltpu.SemaphoreType.DMA,
            pltpu.SemaphoreType.DMA,
            pltpu.VMEM_SHARED((N,), jnp.float32),
            pltpu.VMEM_SHARED((N,), jnp.float32),
            pltpu.VMEM_SHARED((N,), jnp.float32),
            buf(STAGE_PTS),
        ],
        compiler_params=pltpu.CompilerParams(needs_layout_passes=False),
    )
    def kern(rx_hbm, ry_hbm, rz_hbm, ii_hbm, ij_hbm,
             ox_hbm, oy_hbm, oz_hbm, out_hbm, *rest):
        slots = (rest[0:12], rest[12:24])
        sem_ld = rest[24:26]
        sem_ga = rest[26:28]
        sem_out = rest[28:30]
        rx_sh, ry_sh, rz_sh = rest[30:33]
        st_v = rest[33]
        sid = lax.axis_index("s")
        wid = sid * NC + lax.axis_index("c")

        def stage(npts):
            sl = pl.ds(sid * STAGE_PTS, npts)
            sb = pl.ds(0, npts)
            for hbm, sh in ((rx_hbm, rx_sh), (ry_hbm, ry_sh), (rz_hbm, rz_sh)):
                pltpu.sync_copy(hbm.at[sl], st_v.at[sb])
                pltpu.sync_copy(st_v.at[sb], sh.at[sl])

        @pl.when(sid < NS - 1)
        def _():
            stage(STAGE_PTS)

        @pl.when(sid == NS - 1)
        def _():
            stage(stage_tail)

        plsc.subcore_barrier()

        def chunk_id(t):
            return wid + NW * t

        def issue_loads(k, p):
            ii_v, ij_v, ox_v, oy_v, oz_v = slots[p][0:5]
            base = k * CHUNK
            sl = pl.ds(base, CHUNK)
            pltpu.async_copy(ii_hbm.at[sl], ii_v, sem_ld[p])
            pltpu.async_copy(ij_hbm.at[sl], ij_v, sem_ld[p])
            pltpu.async_copy(ox_hbm.at[sl], ox_v, sem_ld[p])
            pltpu.async_copy(oy_hbm.at[sl], oy_v, sem_ld[p])
            pltpu.async_copy(oz_hbm.at[sl], oz_v, sem_ld[p])

        def wait_loads(p):
            for dst in slots[p][0:5]:
                pltpu.make_async_copy(out_hbm.at[pl.ds(0, CHUNK)], dst,
                                      sem_ld[p]).wait()

        def issue_gathers(p):
            (ii_v, ij_v, _ox, _oy, _oz,
             rix, riy, riz, rjx, rjy, rjz, _o) = slots[p]
            for tab, idx_v, dst in ((rx_sh, ii_v, rix),
                                    (ry_sh, ii_v, riy),
                                    (rz_sh, ii_v, riz),
                                    (rx_sh, ij_v, rjx),
                                    (ry_sh, ij_v, rjy),
                                    (rz_sh, ij_v, rjz)):
                pltpu.async_copy(tab.at[idx_v], dst, sem_ga[p])

        def wait_gathers(p):
            for dst in slots[p][5:11]:
                pltpu.make_async_copy(out_hbm.at[pl.ds(0, CHUNK)],
                                      dst, sem_ga[p]).wait()

        def compute(k, p):
            (_, _, ox_v, oy_v, oz_v,
             rix, riy, riz, rjx, rjy, rjz, out_v) = slots[p]
            for g in range(CHUNK // 16):
                sl = pl.ds(16 * g, 16)
                acc = jnp.zeros((16,), jnp.float32)
                for iv, jv, ov in ((rix, rjx, ox_v),
                                   (riy, rjy, oy_v),
                                   (riz, rjz, oz_v)):
                    d = iv[sl] - jv[sl] - ov[sl]
                    acc = acc + d * d
                out_v[sl] = _newton_sqrt(acc)
            pltpu.async_copy(out_v, out_hbm.at[pl.ds(k * CHUNK, CHUNK)],
                             sem_out[p])

        def wait_out(p):
            out_v = slots[p][11]
            pltpu.make_async_copy(out_hbm.at[pl.ds(0, CHUNK)], out_v,
                                  sem_out[p]).wait()

        issue_loads(chunk_id(0), 0)
        wait_loads(0)
        issue_gathers(0)

        def do_trip(t, p):
            k = chunk_id(t)
            knext = chunk_id(t + 1)
            nvalid = knext < nchunks

            @pl.when(nvalid)
            def _():
                issue_loads(knext, 1 - p)

            wait_gathers(p)

            @pl.when(nvalid)
            def _():
                wait_loads(1 - p)
                issue_gathers(1 - p)

            @pl.when(t >= 2)
            def _():
                wait_out(p)
            compute(k, p)

        def body(u, carry):
            t0 = u * 2

            @pl.when(chunk_id(t0) < nchunks)
            def _():
                do_trip(t0, 0)

            @pl.when(chunk_id(t0 + 1) < nchunks)
            def _():
                do_trip(t0 + 1, 1)
            return carry

        lax.fori_loop(0, (ntrips_max + 1) // 2, body, 0)
        pltpu.make_async_copy(out_hbm.at[pl.ds(0, CHUNK)], slots[0][11],
                              sem_out[0]).wait()
        pltpu.make_async_copy(out_hbm.at[pl.ds(0, CHUNK)], slots[1][11],
                              sem_out[1]).wait()

    return kern


def kernel(r, offsets, idx_ik, idx_jk):
    B, N, _ = r.shape
    E = idx_ik.shape[1]
    out = _make_kernel(E, N)(r[0, :, 0], r[0, :, 1], r[0, :, 2],
                             idx_ik[0], idx_jk[0],
                             offsets[0, :, 0], offsets[0, :, 1],
                             offsets[0, :, 2])
    return out.reshape(B, E, 1)


# speedup vs baseline: 102.9233x; 1.0444x over previous
import functools

import jax
import jax.numpy as jnp
from jax import lax
from jax.experimental import pallas as pl
from jax.experimental.pallas import tpu as pltpu
from jax.experimental.pallas import tpu_sc as plsc

NC = 2
NS = 16
NW = NC * NS
CHUNK = 1024
NEWTON_ITERS = 2
STAGE_PTS = 6256


def _newton_sqrt(x):
    xi = lax.bitcast_convert_type(x, jnp.int32)
    yi = jnp.int32(0x5F3759DF) - lax.shift_right_arithmetic(xi, 1)
    y = lax.bitcast_convert_type(yi, jnp.float32)
    half_x = 0.5 * x
    for _ in range(NEWTON_ITERS):
        y = y * (1.5 - half_x * y * y)
    return x * y


def _make_kernel(E, N):
    nchunks = E // CHUNK
    assert nchunks * CHUNK == E
    ntrips_max = -(-nchunks // NW)
    stage_tail = N - (NS - 1) * STAGE_PTS
    assert 0 < stage_tail <= STAGE_PTS
    mesh = plsc.VectorSubcoreMesh(core_axis_name="c", subcore_axis_name="s")

    buf = lambda n, dt=jnp.float32: pltpu.VMEM((n,), dt)
    slot_types = [
        buf(CHUNK, jnp.int32),
        buf(CHUNK, jnp.int32),
        buf(CHUNK), buf(CHUNK), buf(CHUNK),
        buf(CHUNK), buf(CHUNK), buf(CHUNK),
        buf(CHUNK), buf(CHUNK), buf(CHUNK),
        buf(CHUNK),
    ]

    @functools.partial(
        pl.kernel,
        out_type=jax.ShapeDtypeStruct((E,), jnp.float32),
        mesh=mesh,
        scratch_types=slot_types + slot_types + [
            pltpu.SemaphoreType.DMA,
            pltpu.SemaphoreType.DMA,
            pltpu.SemaphoreType.DMA,
            pltpu.SemaphoreType.DMA,
            pltpu.SemaphoreType.DMA,
            pltpu.SemaphoreType.DMA,
            pltpu.VMEM_SHARED((N,), jnp.float32),
            pltpu.VMEM_SHARED((N,), jnp.float32),
            pltpu.VMEM_SHARED((N,), jnp.float32),
            buf(STAGE_PTS),
        ],
        compiler_params=pltpu.CompilerParams(needs_layout_passes=False),
    )
    def kern(rx_hbm, ry_hbm, rz_hbm, ii_hbm, ij_hbm,
             ox_hbm, oy_hbm, oz_hbm, out_hbm, *rest):
        slots = (rest[0:12], rest[12:24])
        sem_ld = rest[24:26]
        sem_ga = rest[26:28]
        sem_out = rest[28:30]
        rx_sh, ry_sh, rz_sh = rest[30:33]
        st_v = rest[33]
        sid = lax.axis_index("s")
        wid = sid * NC + lax.axis_index("c")

        def stage(npts):
            sl = pl.ds(sid * STAGE_PTS, npts)
            sb = pl.ds(0, npts)
            for hbm, sh in ((rx_hbm, rx_sh), (ry_hbm, ry_sh), (rz_hbm, rz_sh)):
                pltpu.sync_copy(hbm.at[sl], st_v.at[sb])
                pltpu.sync_copy(st_v.at[sb], sh.at[sl])

        @pl.when(sid < NS - 1)
        def _():
            stage(STAGE_PTS)

        @pl.when(sid == NS - 1)
        def _():
            stage(stage_tail)

        plsc.subcore_barrier()

        def chunk_id(t):
            return wid + NW * t

        def issue_loads(k, p):
            ii_v, ij_v, ox_v, oy_v, oz_v = slots[p][0:5]
            base = k * CHUNK
            sl = pl.ds(base, CHUNK)
            pltpu.async_copy(ii_hbm.at[sl], ii_v, sem_ld[p])
            pltpu.async_copy(ij_hbm.at[sl], ij_v, sem_ld[p])
            pltpu.async_copy(ox_hbm.at[sl], ox_v, sem_ld[p])
            pltpu.async_copy(oy_hbm.at[sl], oy_v, sem_ld[p])
            pltpu.async_copy(oz_hbm.at[sl], oz_v, sem_ld[p])

        def wait_loads(p):
            for dst in slots[p][0:5]:
                pltpu.make_async_copy(out_hbm.at[pl.ds(0, CHUNK)], dst,
                                      sem_ld[p]).wait()

        def issue_gathers(p):
            (ii_v, ij_v, _ox, _oy, _oz,
             rix, riy, riz, rjx, rjy, rjz, _o) = slots[p]
            for tab, idx_v, dst in ((rx_sh, ii_v, rix),
                                    (ry_sh, ii_v, riy),
                                    (rz_sh, ii_v, riz),
                                    (rx_sh, ij_v, rjx),
                                    (ry_sh, ij_v, rjy),
                                    (rz_sh, ij_v, rjz)):
                pltpu.async_copy(tab.at[idx_v], dst, sem_ga[p])

        def wait_gathers(p):
            for dst in slots[p][5:11]:
                pltpu.make_async_copy(out_hbm.at[pl.ds(0, CHUNK)],
                                      dst, sem_ga[p]).wait()

        def compute(k, p):
            (_, _, ox_v, oy_v, oz_v,
             rix, riy, riz, rjx, rjy, rjz, out_v) = slots[p]
            for g in range(CHUNK // 16):
                sl = pl.ds(16 * g, 16)
                acc = jnp.zeros((16,), jnp.float32)
                for iv, jv, ov in ((rix, rjx, ox_v),
                                   (riy, rjy, oy_v),
                                   (riz, rjz, oz_v)):
                    d = iv[sl] - jv[sl] - ov[sl]
                    acc = acc + d * d
                out_v[sl] = _newton_sqrt(acc)
            pltpu.async_copy(out_v, out_hbm.at[pl.ds(k * CHUNK, CHUNK)],
                             sem_out[p])

        def wait_out(p):
            out_v = slots[p][11]
            pltpu.make_async_copy(out_hbm.at[pl.ds(0, CHUNK)], out_v,
                                  sem_out[p]).wait()

        issue_loads(chunk_id(0), 0)
        wait_loads(0)
        issue_gathers(0)

        def do_trip(t, p):
            k = chunk_id(t)
            knext = chunk_id(t + 1)
            nvalid = knext < nchunks

            @pl.when(nvalid)
            def _():
                issue_loads(knext, 1 - p)

            wait_gathers(p)

            @pl.when(nvalid)
            def _():
                wait_loads(1 - p)
                issue_gathers(1 - p)

            @pl.when(t >= 2)
            def _():
                wait_out(p)
            compute(k, p)

        def body(u, carry):
            t0 = u * 2

            @pl.when(chunk_id(t0) < nchunks)
            def _():
                do_trip(t0, 0)

            @pl.when(chunk_id(t0 + 1) < nchunks)
            def _():
                do_trip(t0 + 1, 1)
            return carry

        lax.fori_loop(0, (ntrips_max + 1) // 2, body, 0)
        pltpu.make_async_copy(out_hbm.at[pl.ds(0, CHUNK)], slots[0][11],
                              sem_out[0]).wait()
        pltpu.make_async_copy(out_hbm.at[pl.ds(0, CHUNK)], slots[1][11],
                              sem_out[1]).wait()

    return kern


def kernel(r, offsets, idx_ik, idx_jk):
    B, N, _ = r.shape
    E = idx_ik.shape[1]
    out = _make_kernel(E, N)(r[0, :, 0], r[0, :, 1], r[0, :, 2],
                             idx_ik[0], idx_jk[0],
                             offsets[0, :, 0], offsets[0, :, 1],
                             offsets[0, :, 2])
    return out.reshape(B, E, 1)
